# Initial kernel scaffold; baseline (speedup 1.0000x reference)
import jax
import jax.numpy as jnp
from jax.experimental import pallas as pl


def kernel(input, segs):
    raise NotImplementedError("write your pallas kernel here")


# speedup vs baseline: 17.6332x; 17.6332x over previous
import jax
import jax.numpy as jnp
from jax.experimental import pallas as pl
from jax.experimental.pallas import tpu as pltpu

K = 16
ROW_BLOCK = 256


def _knn_block_kernel(rows_ref, pts_ref, dist_ref, idx_ref):
    rows = rows_ref[...]
    pts = pts_ref[...]
    r = rows.shape[0]
    n = pts.shape[0]

    sq_r = jnp.sum(rows * rows, axis=1, keepdims=True)
    sq_p = jnp.sum(pts * pts, axis=1, keepdims=True)
    dot = jax.lax.dot_general(
        rows, pts, (((1,), (1,)), ((), ())),
        preferred_element_type=jnp.float32)
    d2 = sq_r + sq_p.reshape(1, n) - 2.0 * dot
    d2 = jnp.maximum(d2, 0.0)

    iota = jax.lax.broadcasted_iota(jnp.int32, (r, n), 1)
    big_idx = jnp.int32(n)
    inf = jnp.float32(jnp.inf)

    dist_cols = []
    idx_cols = []
    for _ in range(K):
        vmin = jnp.min(d2, axis=1, keepdims=True)
        imin = jnp.min(jnp.where(d2 == vmin, iota, big_idx),
                       axis=1, keepdims=True)
        d2 = jnp.where(iota == imin, inf, d2)
        dist_cols.append(vmin)
        idx_cols.append(imin)

    dist_ref[...] = jnp.concatenate(dist_cols, axis=1)
    idx_ref[...] = jnp.concatenate(idx_cols, axis=1)


def kernel(input, segs):
    m, d = input.shape
    nseg = segs.shape[0]
    n = m // nseg
    nb = n // ROW_BLOCK

    grid = (nseg, nb)
    dist, idx = pl.pallas_call(
        _knn_block_kernel,
        grid=grid,
        in_specs=[
            pl.BlockSpec((ROW_BLOCK, d), lambda s, b: (s * nb + b, 0)),
            pl.BlockSpec((n, d), lambda s, b: (s, 0)),
        ],
        out_specs=[
            pl.BlockSpec((ROW_BLOCK, K), lambda s, b: (s * nb + b, 0)),
            pl.BlockSpec((ROW_BLOCK, K), lambda s, b: (s * nb + b, 0)),
        ],
        out_shape=[
            jax.ShapeDtypeStruct((m, K), jnp.float32),
            jax.ShapeDtypeStruct((m, K), jnp.int32),
        ],
        compiler_params=pltpu.CompilerParams(
            dimension_semantics=("arbitrary", "arbitrary"),
        ),
    )(input, input)

    offsets = jnp.concatenate(
        [jnp.zeros((1,), dtype=segs.dtype), jnp.cumsum(segs)])
    row_off = jnp.repeat(offsets[:-1], n)
    src = (idx + row_off[:, None]).astype(jnp.int64).reshape(-1)
    dst = jnp.repeat(jnp.arange(n, dtype=jnp.int64)[None, :]
                     + offsets[:-1][:, None].astype(jnp.int64), K).reshape(-1)
    return src, dst, dist
